# baseline (speedup 1.0000x reference)
import functools

import jax
import jax.numpy as jnp
from jax import lax
from jax.experimental import pallas as pl
from jax.experimental.pallas import tpu as pltpu
from jax.experimental.pallas import tpu_sc as plsc

N = 10000
D = 128
M = 64
DW = 128
CH = 128
NC = 2
NS = 16
NW = NC * NS
BN = 1000
BE = 4096
NSEG = 10112
RPT = NSEG // NS


def _silu(v):
    return v * jax.nn.sigmoid(v)


def _dg(a, b):
    return lax.dot_general(a, b, (((1,), (1,)), ((), ())),
                           preferred_element_type=jnp.float32)


def _table_body(h_ref, xp_ref, wr_ref, wc_ref, rt_ref, ct_ref):
    hh = h_ref[...]
    xp = xp_ref[...]
    rt_ref[...] = jnp.concatenate([_dg(hh, wr_ref[...]), xp], axis=1)
    ct_ref[...] = jnp.concatenate([_dg(hh, wc_ref[...]), -xp], axis=1)


def _edge_body(u_ref, ea_ref, wrad_ref, wea_ref, be1_ref, we2_ref, be2_ref,
               watt_ref, batt_ref, out_ref):
    u = u_ref[...].astype(jnp.float32)
    pq = u[:, :M]
    xd = u[:, M:DW]
    s2b = _dg(xd * xd, jnp.ones((M, M), jnp.float32))
    radb = jnp.sqrt(s2b)
    z = _silu(pq + radb * wrad_ref[...] + _dg(ea_ref[...], wea_ref[...])
              + be1_ref[...])
    m = _silu(_dg(z, we2_ref[...]) + be2_ref[...])
    att = jax.nn.sigmoid(_dg(m, watt_ref[...]) + batt_ref[...])
    msg = m * att
    out_ref[...] = jnp.concatenate([msg, jnp.zeros_like(msg)], axis=1)


def _node_body(h_ref, ms_ref, wh1a_ref, wh1b_ref, bh1_ref, wh2_ref, bh2_ref,
               out_ref):
    hh = h_ref[...]
    ms = ms_ref[0][:, :M] + ms_ref[1][:, :M]
    t = _silu(_dg(hh, wh1a_ref[...]) + _dg(ms, wh1b_ref[...]) + bh1_ref[...])
    out_ref[...] = hh + _dg(t, wh2_ref[...]) + bh2_ref[...]


NBUF = 3


def _gather_body(K, rt_hbm, ct_hbm, rowg_hbm, colg_hbm, u_hbm,
                 rowv, colv, rbufs, cbufs, semr, semc, semw):
    c = lax.axis_index("c")
    s = lax.axis_index("s")
    wid = c * NS + s
    base = wid * (K * CH)
    pltpu.sync_copy(rowg_hbm.at[wid], rowv)
    pltpu.sync_copy(colg_hbm.at[wid], colv)

    def _add(rb, cb):
        def body(i, _):
            for j in range(DW // 16):
                sl = pl.ds(j * 16, 16)
                rb[i, sl] = rb[i, sl] + cb[i, sl]
            return 0
        lax.fori_loop(0, CH, body, 0, unroll=2)

    def _issue(k, i):
        pltpu.async_copy(rt_hbm.at[rowv.at[k]], rbufs[i], semr[i])
        pltpu.async_copy(ct_hbm.at[colv.at[k]], cbufs[i], semc[i])

    def _waitg(k, i):
        pltpu.make_async_copy(rt_hbm.at[rowv.at[k]], rbufs[i], semr[i]).wait()
        pltpu.make_async_copy(ct_hbm.at[colv.at[k]], cbufs[i], semc[i]).wait()

    def _emit(k, i):
        pltpu.async_copy(rbufs[i], u_hbm.at[pl.ds(base + k * CH, CH)], semw[i])

    def _waitw(k, i):
        pltpu.make_async_copy(rbufs[i], u_hbm.at[pl.ds(base + k * CH, CH)],
                              semw[i]).wait()

    for i in range(NBUF):
        _issue(i, i)

    def loop_body(t, _):
        k0 = NBUF * t
        for i in range(NBUF):
            _waitg(k0 + i, i)
            _add(rbufs[i], cbufs[i])
            _emit(k0 + i, i)
        for i in range(NBUF):
            _waitw(k0 + i, i)
            _issue(k0 + NBUF + i, i)
        return 0

    lax.fori_loop(0, K // NBUF - 1, loop_body, 0)
    kl = K - NBUF
    for i in range(NBUF):
        _waitg(kl + i, i)
        _add(rbufs[i], cbufs[i])
        _emit(kl + i, i)
    for i in range(NBUF):
        _waitw(kl + i, i)


_RPT_CHUNKS = [(o, min(128, RPT - o)) for o in range(0, RPT, 128)]


def _scatter_body(K, msg_hbm, sidx_hbm, out_hbm, sidxv, mbuf, msum_sh):
    c = lax.axis_index("c")
    s = lax.axis_index("s")
    wid = c * NS + s
    base = wid * (K * CH)
    pltpu.sync_copy(sidx_hbm.at[wid], sidxv)

    def zrow(i, _):
        for j in range(DW // 16):
            mbuf[i, pl.ds(j * 16, 16)] = jnp.zeros((16,), jnp.float32)
        return 0
    lax.fori_loop(0, CH, zrow, 0)
    for off, sz in _RPT_CHUNKS:
        pltpu.sync_copy(mbuf.at[pl.ds(0, sz)],
                        msum_sh.at[pl.ds(s * RPT + off, sz)])
    plsc.subcore_barrier()

    def body(k, _):
        pltpu.sync_copy(msg_hbm.at[pl.ds(base + k * CH, CH)], mbuf)
        pltpu.sync_copy(mbuf, msum_sh.at[sidxv.at[k]], add=True)
        return 0
    lax.fori_loop(0, K, body, 0)
    plsc.subcore_barrier()

    for off, sz in _RPT_CHUNKS:
        pltpu.sync_copy(msum_sh.at[pl.ds(s * RPT + off, sz)],
                        mbuf.at[pl.ds(0, sz)])
        pltpu.sync_copy(mbuf.at[pl.ds(0, sz)],
                        out_hbm.at[c, pl.ds(s * RPT + off, sz)])


def kernel(x, h, edge_index, edge_attr, W_e1, b_e1, W_e2, b_e2, W_att, b_att,
           W_x1, b_x1, W_x2, b_x2, W_h1, b_h1, W_h2, b_h2):
    E = edge_index.shape[1]
    K = -(-E // (NW * CH))
    K = -(-K // NBUF) * NBUF
    E_pad = NW * K * CH
    KE = E_pad // BE
    f32 = jnp.float32

    row = edge_index[0]
    col = edge_index[1]
    pad = E_pad - E
    row_g = jnp.concatenate([row, jnp.zeros((pad,), jnp.int32)]).reshape(NW, K, CH)
    col_g = jnp.concatenate([col, jnp.zeros((pad,), jnp.int32)]).reshape(NW, K, CH)
    sidx = jnp.concatenate([row, jnp.full((pad,), N, jnp.int32)]).reshape(NW, K, CH)
    ea8 = jnp.concatenate([edge_attr, jnp.zeros((pad, 4), f32)])
    ea8 = jnp.pad(ea8, ((0, 0), (0, 4)))
    xp = jnp.pad(x, ((0, 0), (0, M - 3)))

    wr = W_e1[:, :D]
    wc = W_e1[:, D:2 * D]
    wrad = W_e1[:, 2 * D:2 * D + 1].reshape(1, M)
    wea = jnp.pad(W_e1[:, 2 * D + 1:], ((0, 0), (0, 4)))
    be1 = b_e1.reshape(1, M)
    be2 = b_e2.reshape(1, M)
    watt_rep = jnp.tile(W_att, (M, 1))
    batt_rep = jnp.tile(b_att.reshape(1, 1), (1, M))
    wh1a = W_h1[:, :D]
    wh1b = W_h1[:, D:]
    bh1 = b_h1.reshape(1, M)
    bh2 = b_h2.reshape(1, D)

    rt, ct = pl.pallas_call(
        _table_body,
        grid=(N // BN,),
        in_specs=[
            pl.BlockSpec((BN, D), lambda i: (i, 0)),
            pl.BlockSpec((BN, M), lambda i: (i, 0)),
            pl.BlockSpec((M, D), lambda i: (0, 0)),
            pl.BlockSpec((M, D), lambda i: (0, 0)),
        ],
        out_specs=[
            pl.BlockSpec((BN, DW), lambda i: (i, 0)),
            pl.BlockSpec((BN, DW), lambda i: (i, 0)),
        ],
        out_shape=[
            jax.ShapeDtypeStruct((N, DW), f32),
            jax.ShapeDtypeStruct((N, DW), f32),
        ],
    )(h, xp, wr, wc)

    mesh = plsc.VectorSubcoreMesh(core_axis_name="c", subcore_axis_name="s")
    u = pl.kernel(
        functools.partial(_gather_body, K),
        out_type=jax.ShapeDtypeStruct((E_pad, DW), f32),
        mesh=mesh,
        scratch_types=[
            pltpu.VMEM((K, CH), jnp.int32),
            pltpu.VMEM((K, CH), jnp.int32),
            [pltpu.VMEM((CH, DW), f32) for _ in range(NBUF)],
            [pltpu.VMEM((CH, DW), f32) for _ in range(NBUF)],
            [pltpu.SemaphoreType.DMA for _ in range(NBUF)],
            [pltpu.SemaphoreType.DMA for _ in range(NBUF)],
            [pltpu.SemaphoreType.DMA for _ in range(NBUF)],
        ],
    )(rt, ct, row_g, col_g)

    msg = pl.pallas_call(
        _edge_body,
        grid=(KE,),
        in_specs=[
            pl.BlockSpec((BE, DW), lambda i: (i, 0)),
            pl.BlockSpec((BE, 8), lambda i: (i, 0)),
            pl.BlockSpec((1, M), lambda i: (0, 0)),
            pl.BlockSpec((M, 8), lambda i: (0, 0)),
            pl.BlockSpec((1, M), lambda i: (0, 0)),
            pl.BlockSpec((M, M), lambda i: (0, 0)),
            pl.BlockSpec((1, M), lambda i: (0, 0)),
            pl.BlockSpec((M, M), lambda i: (0, 0)),
            pl.BlockSpec((1, M), lambda i: (0, 0)),
        ],
        out_specs=pl.BlockSpec((BE, DW), lambda i: (i, 0)),
        out_shape=jax.ShapeDtypeStruct((E_pad, DW), f32),
    )(u, ea8, wrad, wea, be1, W_e2, be2, watt_rep, batt_rep)

    msum2 = pl.kernel(
        functools.partial(_scatter_body, K),
        out_type=jax.ShapeDtypeStruct((NC, NSEG, DW), f32),
        mesh=mesh,
        scratch_types=[
            pltpu.VMEM((K, CH), jnp.int32),
            pltpu.VMEM((CH, DW), f32),
            pltpu.VMEM_SHARED((NSEG, DW), f32),
        ],
    )(msg, sidx)

    out = pl.pallas_call(
        _node_body,
        grid=(N // BN,),
        in_specs=[
            pl.BlockSpec((BN, D), lambda i: (i, 0)),
            pl.BlockSpec((NC, BN, DW), lambda i: (0, i, 0)),
            pl.BlockSpec((M, D), lambda i: (0, 0)),
            pl.BlockSpec((M, M), lambda i: (0, 0)),
            pl.BlockSpec((1, M), lambda i: (0, 0)),
            pl.BlockSpec((D, M), lambda i: (0, 0)),
            pl.BlockSpec((1, D), lambda i: (0, 0)),
        ],
        out_specs=pl.BlockSpec((BN, D), lambda i: (i, 0)),
        out_shape=jax.ShapeDtypeStruct((N, D), f32),
    )(h, msum2, wh1a, wh1b, bh1, W_h2, bh2)
    return out


# speedup vs baseline: 1.2476x; 1.2476x over previous
import functools

import jax
import jax.numpy as jnp
from jax import lax
from jax.experimental import pallas as pl
from jax.experimental.pallas import tpu as pltpu
from jax.experimental.pallas import tpu_sc as plsc

N = 10000
D = 128
M = 64
DW = 128
CH = 128
NC = 2
NS = 16
NW = NC * NS
BN = 1000
BE = 4096
NSEG = 10112
RPT = NSEG // NS


def _silu(v):
    return v * jax.nn.sigmoid(v)


def _dg(a, b):
    return lax.dot_general(a, b, (((1,), (1,)), ((), ())),
                           preferred_element_type=jnp.float32)


def _table_body(h_ref, xp_ref, wr_ref, wc_ref, rt_ref, ct_ref):
    hh = h_ref[...]
    xp = xp_ref[...]
    rt_ref[...] = jnp.concatenate([_dg(hh, wr_ref[...]), xp], axis=1)
    ct_ref[...] = jnp.concatenate([_dg(hh, wc_ref[...]), -xp], axis=1)


def _edge_body(u_ref, ea_ref, wrad_ref, wea_ref, be1_ref, we2_ref, be2_ref,
               watt_ref, batt_ref, out_ref):
    u = u_ref[...].astype(jnp.float32)
    pq = u[:, :M]
    xd = u[:, M:DW]
    s2b = _dg(xd * xd, jnp.ones((M, M), jnp.float32))
    radb = jnp.sqrt(s2b)
    z = _silu(pq + radb * wrad_ref[...] + _dg(ea_ref[...], wea_ref[...])
              + be1_ref[...])
    m = _silu(_dg(z, we2_ref[...]) + be2_ref[...])
    att = jax.nn.sigmoid(_dg(m, watt_ref[...]) + batt_ref[...])
    msg = m * att
    out_ref[...] = jnp.concatenate([msg, jnp.zeros_like(msg)], axis=1)


def _node_body(h_ref, ms_ref, wh1a_ref, wh1b_ref, bh1_ref, wh2_ref, bh2_ref,
               out_ref):
    hh = h_ref[...]
    ms = ms_ref[0][:, :M] + ms_ref[1][:, :M]
    t = _silu(_dg(hh, wh1a_ref[...]) + _dg(ms, wh1b_ref[...]) + bh1_ref[...])
    out_ref[...] = hh + _dg(t, wh2_ref[...]) + bh2_ref[...]


NBUF = 2


def _gather_body(K, rt_hbm, ct_hbm, rowg_hbm, colg_hbm, u_hbm,
                 rowv, colv, rbufs, cbufs, semr, semc, semw):
    c = lax.axis_index("c")
    s = lax.axis_index("s")
    wid = c * NS + s
    base = wid * (K * CH)
    pltpu.sync_copy(rowg_hbm.at[wid], rowv)
    pltpu.sync_copy(colg_hbm.at[wid], colv)

    def _add(rb, cb):
        def body(i, _):
            for j in range(DW // 16):
                sl = pl.ds(j * 16, 16)
                rb[i, sl] = rb[i, sl] + cb[i, sl]
            return 0
        lax.fori_loop(0, CH, body, 0, unroll=2)

    def _issue(k, i):
        pltpu.async_copy(rt_hbm.at[rowv.at[k]], rbufs[i], semr[i])
        pltpu.async_copy(ct_hbm.at[colv.at[k]], cbufs[i], semc[i])

    def _waitg(k, i):
        pltpu.make_async_copy(rt_hbm.at[rowv.at[k]], rbufs[i], semr[i]).wait()
        pltpu.make_async_copy(ct_hbm.at[colv.at[k]], cbufs[i], semc[i]).wait()

    def _emit(k, i):
        pltpu.async_copy(rbufs[i], u_hbm.at[pl.ds(base + k * CH, CH)], semw[i])

    def _waitw(k, i):
        pltpu.make_async_copy(rbufs[i], u_hbm.at[pl.ds(base + k * CH, CH)],
                              semw[i]).wait()

    for i in range(NBUF):
        _issue(i, i)

    def loop_body(t, _):
        k0 = NBUF * t
        for i in range(NBUF):
            _waitg(k0 + i, i)
            _add(rbufs[i], cbufs[i])
            _emit(k0 + i, i)
        for i in range(NBUF):
            _waitw(k0 + i, i)
            _issue(k0 + NBUF + i, i)
        return 0

    lax.fori_loop(0, K // NBUF - 1, loop_body, 0)
    kl = K - NBUF
    for i in range(NBUF):
        _waitg(kl + i, i)
        _add(rbufs[i], cbufs[i])
        _emit(kl + i, i)
    for i in range(NBUF):
        _waitw(kl + i, i)


_RPT_CHUNKS = [(o, min(128, RPT - o)) for o in range(0, RPT, 128)]


def _scatter_body(K, msg_hbm, sidx_hbm, out_hbm, sidxv, mbuf, msum_sh):
    c = lax.axis_index("c")
    s = lax.axis_index("s")
    wid = c * NS + s
    base = wid * (K * CH)
    pltpu.sync_copy(sidx_hbm.at[wid], sidxv)

    def zrow(i, _):
        for j in range(DW // 16):
            mbuf[i, pl.ds(j * 16, 16)] = jnp.zeros((16,), jnp.float32)
        return 0
    lax.fori_loop(0, CH, zrow, 0)
    for off, sz in _RPT_CHUNKS:
        pltpu.sync_copy(mbuf.at[pl.ds(0, sz)],
                        msum_sh.at[pl.ds(s * RPT + off, sz)])
    plsc.subcore_barrier()

    def body(k, _):
        pltpu.sync_copy(msg_hbm.at[pl.ds(base + k * CH, CH)], mbuf)
        pltpu.sync_copy(mbuf, msum_sh.at[sidxv.at[k]], add=True)
        return 0
    lax.fori_loop(0, K, body, 0)
    plsc.subcore_barrier()

    for off, sz in _RPT_CHUNKS:
        pltpu.sync_copy(msum_sh.at[pl.ds(s * RPT + off, sz)],
                        mbuf.at[pl.ds(0, sz)])
        pltpu.sync_copy(mbuf.at[pl.ds(0, sz)],
                        out_hbm.at[c, pl.ds(s * RPT + off, sz)])


def kernel(x, h, edge_index, edge_attr, W_e1, b_e1, W_e2, b_e2, W_att, b_att,
           W_x1, b_x1, W_x2, b_x2, W_h1, b_h1, W_h2, b_h2):
    E = edge_index.shape[1]
    K = -(-E // (NW * CH))
    K = -(-K // NBUF) * NBUF
    E_pad = NW * K * CH
    KE = E_pad // BE
    f32 = jnp.float32

    row = edge_index[0]
    col = edge_index[1]
    pad = E_pad - E
    row_g = jnp.concatenate([row, jnp.zeros((pad,), jnp.int32)]).reshape(NW, K, CH)
    col_g = jnp.concatenate([col, jnp.zeros((pad,), jnp.int32)]).reshape(NW, K, CH)
    sidx = jnp.concatenate([row, jnp.full((pad,), N, jnp.int32)]).reshape(NW, K, CH)
    ea8 = jnp.concatenate([edge_attr, jnp.zeros((pad, 4), f32)])
    ea8 = jnp.pad(ea8, ((0, 0), (0, 4)))
    xp = jnp.pad(x, ((0, 0), (0, M - 3)))

    wr = W_e1[:, :D]
    wc = W_e1[:, D:2 * D]
    wrad = W_e1[:, 2 * D:2 * D + 1].reshape(1, M)
    wea = jnp.pad(W_e1[:, 2 * D + 1:], ((0, 0), (0, 4)))
    be1 = b_e1.reshape(1, M)
    be2 = b_e2.reshape(1, M)
    watt_rep = jnp.tile(W_att, (M, 1))
    batt_rep = jnp.tile(b_att.reshape(1, 1), (1, M))
    wh1a = W_h1[:, :D]
    wh1b = W_h1[:, D:]
    bh1 = b_h1.reshape(1, M)
    bh2 = b_h2.reshape(1, D)

    rt, ct = pl.pallas_call(
        _table_body,
        grid=(N // BN,),
        in_specs=[
            pl.BlockSpec((BN, D), lambda i: (i, 0)),
            pl.BlockSpec((BN, M), lambda i: (i, 0)),
            pl.BlockSpec((M, D), lambda i: (0, 0)),
            pl.BlockSpec((M, D), lambda i: (0, 0)),
        ],
        out_specs=[
            pl.BlockSpec((BN, DW), lambda i: (i, 0)),
            pl.BlockSpec((BN, DW), lambda i: (i, 0)),
        ],
        out_shape=[
            jax.ShapeDtypeStruct((N, DW), f32),
            jax.ShapeDtypeStruct((N, DW), f32),
        ],
    )(h, xp, wr, wc)

    mesh = plsc.VectorSubcoreMesh(core_axis_name="c", subcore_axis_name="s")
    u = pl.kernel(
        functools.partial(_gather_body, K),
        out_type=jax.ShapeDtypeStruct((E_pad, DW), f32),
        mesh=mesh,
        scratch_types=[
            pltpu.VMEM((K, CH), jnp.int32),
            pltpu.VMEM((K, CH), jnp.int32),
            [pltpu.VMEM((CH, DW), f32) for _ in range(NBUF)],
            [pltpu.VMEM((CH, DW), f32) for _ in range(NBUF)],
            [pltpu.SemaphoreType.DMA for _ in range(NBUF)],
            [pltpu.SemaphoreType.DMA for _ in range(NBUF)],
            [pltpu.SemaphoreType.DMA for _ in range(NBUF)],
        ],
    )(rt, ct, row_g, col_g)

    msg = pl.pallas_call(
        _edge_body,
        grid=(KE,),
        in_specs=[
            pl.BlockSpec((BE, DW), lambda i: (i, 0)),
            pl.BlockSpec((BE, 8), lambda i: (i, 0)),
            pl.BlockSpec((1, M), lambda i: (0, 0)),
            pl.BlockSpec((M, 8), lambda i: (0, 0)),
            pl.BlockSpec((1, M), lambda i: (0, 0)),
            pl.BlockSpec((M, M), lambda i: (0, 0)),
            pl.BlockSpec((1, M), lambda i: (0, 0)),
            pl.BlockSpec((M, M), lambda i: (0, 0)),
            pl.BlockSpec((1, M), lambda i: (0, 0)),
        ],
        out_specs=pl.BlockSpec((BE, DW), lambda i: (i, 0)),
        out_shape=jax.ShapeDtypeStruct((E_pad, DW), f32),
    )(u, ea8, wrad, wea, be1, W_e2, be2, watt_rep, batt_rep)

    msum2 = pl.kernel(
        functools.partial(_scatter_body, K),
        out_type=jax.ShapeDtypeStruct((NC, NSEG, DW), f32),
        mesh=mesh,
        scratch_types=[
            pltpu.VMEM((K, CH), jnp.int32),
            pltpu.VMEM((CH, DW), f32),
            pltpu.VMEM_SHARED((NSEG, DW), f32),
        ],
    )(msg, sidx)

    out = pl.pallas_call(
        _node_body,
        grid=(N // BN,),
        in_specs=[
            pl.BlockSpec((BN, D), lambda i: (i, 0)),
            pl.BlockSpec((NC, BN, DW), lambda i: (0, i, 0)),
            pl.BlockSpec((M, D), lambda i: (0, 0)),
            pl.BlockSpec((M, M), lambda i: (0, 0)),
            pl.BlockSpec((1, M), lambda i: (0, 0)),
            pl.BlockSpec((D, M), lambda i: (0, 0)),
            pl.BlockSpec((1, D), lambda i: (0, 0)),
        ],
        out_specs=pl.BlockSpec((BN, D), lambda i: (i, 0)),
        out_shape=jax.ShapeDtypeStruct((N, D), f32),
    )(h, msum2, wh1a, wh1b, bh1, W_h2, bh2)
    return out


# speedup vs baseline: 1.7086x; 1.3695x over previous
import functools

import jax
import jax.numpy as jnp
from jax import lax
from jax.experimental import pallas as pl
from jax.experimental.pallas import tpu as pltpu
from jax.experimental.pallas import tpu_sc as plsc

N = 10000
D = 128
M = 64
DW = 128
CH = 128
NC = 2
NS = 16
NW = NC * NS
BN = 1000
BE = 4096
NSEG = 10112
RPT = NSEG // NS


def _silu(v):
    return v * jax.nn.sigmoid(v)


def _dg(a, b):
    return lax.dot_general(a, b, (((1,), (1,)), ((), ())),
                           preferred_element_type=jnp.float32)


def _table_body(h_ref, xp_ref, wr_ref, wc_ref, rt_ref, ct_ref):
    hh = h_ref[...]
    xp = xp_ref[...]
    rt_ref[...] = jnp.concatenate([_dg(hh, wr_ref[...]), xp], axis=1)
    ct_ref[...] = jnp.concatenate([_dg(hh, wc_ref[...]), -xp], axis=1)


def _edge_body(u_ref, ea_ref, wrad_ref, wea_ref, be1_ref, we2_ref, be2_ref,
               watt_ref, batt_ref, out_ref):
    u = u_ref[...].astype(jnp.float32)
    pq = u[:, :M]
    xd = u[:, M:DW]
    s2b = _dg(xd * xd, jnp.ones((M, M), jnp.float32))
    radb = jnp.sqrt(s2b)
    z = _silu(pq + radb * wrad_ref[...] + _dg(ea_ref[...], wea_ref[...])
              + be1_ref[...])
    m = _silu(_dg(z, we2_ref[...]) + be2_ref[...])
    att = jax.nn.sigmoid(_dg(m, watt_ref[...]) + batt_ref[...])
    msg = m * att
    out_ref[...] = jnp.concatenate([msg, jnp.zeros_like(msg)], axis=1)


def _node_body(h_ref, ms_ref, wh1a_ref, wh1b_ref, bh1_ref, wh2_ref, bh2_ref,
               out_ref):
    hh = h_ref[...]
    ms = ms_ref[0][:, :M] + ms_ref[1][:, :M]
    t = _silu(_dg(hh, wh1a_ref[...]) + _dg(ms, wh1b_ref[...]) + bh1_ref[...])
    out_ref[...] = hh + _dg(t, wh2_ref[...]) + bh2_ref[...]


NBUF = 2


def _gather_body(K, rt_hbm, ct_hbm, rowg_hbm, colg_hbm, u_hbm,
                 rowv, colv, rbufs, cbufs, semr, semc, semw):
    c = lax.axis_index("c")
    s = lax.axis_index("s")
    wid = c * NS + s
    base = wid * (K * CH)
    pltpu.sync_copy(rowg_hbm.at[wid], rowv)
    pltpu.sync_copy(colg_hbm.at[wid], colv)

    def _add(rb, cb):
        def body(i, _):
            for j in range(DW // 16):
                sl = pl.ds(j * 16, 16)
                rb[i, sl] = rb[i, sl] + cb[i, sl]
            return 0
        lax.fori_loop(0, CH, body, 0, unroll=2)

    def _issue(k, i):
        pltpu.async_copy(rt_hbm.at[rowv.at[k]], rbufs[i], semr[i])
        pltpu.async_copy(ct_hbm.at[colv.at[k]], cbufs[i], semc[i])

    def _waitg(k, i):
        pltpu.make_async_copy(rt_hbm.at[rowv.at[k]], rbufs[i], semr[i]).wait()
        pltpu.make_async_copy(ct_hbm.at[colv.at[k]], cbufs[i], semc[i]).wait()

    def _emit(k, i):
        pltpu.async_copy(rbufs[i], u_hbm.at[pl.ds(base + k * CH, CH)], semw[i])

    def _waitw(k, i):
        pltpu.make_async_copy(rbufs[i], u_hbm.at[pl.ds(base + k * CH, CH)],
                              semw[i]).wait()

    def _emit_sync(k, i):
        pltpu.sync_copy(rbufs[i], u_hbm.at[pl.ds(base + k * CH, CH)])

    _issue(0, 0)

    def loop_body(k2, _):
        k0 = 2 * k2
        _issue(k0 + 1, 1)
        _waitg(k0, 0)
        _add(rbufs[0], cbufs[0])
        _emit_sync(k0, 0)
        _issue(k0 + 2, 0)
        _waitg(k0 + 1, 1)
        _add(rbufs[1], cbufs[1])
        _emit_sync(k0 + 1, 1)
        return 0

    lax.fori_loop(0, (K - 1) // 2, loop_body, 0)
    kl = K - 1
    _waitg(kl, 0)
    _add(rbufs[0], cbufs[0])
    _emit_sync(kl, 0)


_RPT_CHUNKS = [(o, min(128, RPT - o)) for o in range(0, RPT, 128)]


def _scatter_body(K, msg_hbm, sidx_hbm, out_hbm, sidxv, mbuf, msum_sh):
    c = lax.axis_index("c")
    s = lax.axis_index("s")
    wid = c * NS + s
    base = wid * (K * CH)
    pltpu.sync_copy(sidx_hbm.at[wid], sidxv)

    def zrow(i, _):
        for j in range(DW // 16):
            mbuf[i, pl.ds(j * 16, 16)] = jnp.zeros((16,), jnp.float32)
        return 0
    lax.fori_loop(0, CH, zrow, 0)
    for off, sz in _RPT_CHUNKS:
        pltpu.sync_copy(mbuf.at[pl.ds(0, sz)],
                        msum_sh.at[pl.ds(s * RPT + off, sz)])
    plsc.subcore_barrier()

    def body(k, _):
        pltpu.sync_copy(msg_hbm.at[pl.ds(base + k * CH, CH)], mbuf)
        pltpu.sync_copy(mbuf, msum_sh.at[sidxv.at[k]], add=True)
        return 0
    lax.fori_loop(0, K, body, 0)
    plsc.subcore_barrier()

    for off, sz in _RPT_CHUNKS:
        pltpu.sync_copy(msum_sh.at[pl.ds(s * RPT + off, sz)],
                        mbuf.at[pl.ds(0, sz)])
        pltpu.sync_copy(mbuf.at[pl.ds(0, sz)],
                        out_hbm.at[c, pl.ds(s * RPT + off, sz)])


def kernel(x, h, edge_index, edge_attr, W_e1, b_e1, W_e2, b_e2, W_att, b_att,
           W_x1, b_x1, W_x2, b_x2, W_h1, b_h1, W_h2, b_h2):
    E = edge_index.shape[1]
    K = -(-E // (NW * CH))
    if K % 2 == 0:
        K += 1
    E_pad = NW * K * CH
    KE = E_pad // BE
    f32 = jnp.float32

    row = edge_index[0]
    col = edge_index[1]
    pad = E_pad - E
    row_g = jnp.concatenate([row, jnp.zeros((pad,), jnp.int32)]).reshape(NW, K, CH)
    col_g = jnp.concatenate([col, jnp.zeros((pad,), jnp.int32)]).reshape(NW, K, CH)
    sidx = jnp.concatenate([row, jnp.full((pad,), N, jnp.int32)]).reshape(NW, K, CH)
    ea8 = jnp.concatenate([edge_attr, jnp.zeros((pad, 4), f32)])
    ea8 = jnp.pad(ea8, ((0, 0), (0, 4)))
    xp = jnp.pad(x, ((0, 0), (0, M - 3)))

    wr = W_e1[:, :D]
    wc = W_e1[:, D:2 * D]
    wrad = W_e1[:, 2 * D:2 * D + 1].reshape(1, M)
    wea = jnp.pad(W_e1[:, 2 * D + 1:], ((0, 0), (0, 4)))
    be1 = b_e1.reshape(1, M)
    be2 = b_e2.reshape(1, M)
    watt_rep = jnp.tile(W_att, (M, 1))
    batt_rep = jnp.tile(b_att.reshape(1, 1), (1, M))
    wh1a = W_h1[:, :D]
    wh1b = W_h1[:, D:]
    bh1 = b_h1.reshape(1, M)
    bh2 = b_h2.reshape(1, D)

    rt, ct = pl.pallas_call(
        _table_body,
        grid=(N // BN,),
        in_specs=[
            pl.BlockSpec((BN, D), lambda i: (i, 0)),
            pl.BlockSpec((BN, M), lambda i: (i, 0)),
            pl.BlockSpec((M, D), lambda i: (0, 0)),
            pl.BlockSpec((M, D), lambda i: (0, 0)),
        ],
        out_specs=[
            pl.BlockSpec((BN, DW), lambda i: (i, 0)),
            pl.BlockSpec((BN, DW), lambda i: (i, 0)),
        ],
        out_shape=[
            jax.ShapeDtypeStruct((N, DW), f32),
            jax.ShapeDtypeStruct((N, DW), f32),
        ],
    )(h, xp, wr, wc)

    mesh = plsc.VectorSubcoreMesh(core_axis_name="c", subcore_axis_name="s")
    u = pl.kernel(
        functools.partial(_gather_body, K),
        out_type=jax.ShapeDtypeStruct((E_pad, DW), f32),
        mesh=mesh,
        scratch_types=[
            pltpu.VMEM((K, CH), jnp.int32),
            pltpu.VMEM((K, CH), jnp.int32),
            [pltpu.VMEM((CH, DW), f32) for _ in range(NBUF)],
            [pltpu.VMEM((CH, DW), f32) for _ in range(NBUF)],
            [pltpu.SemaphoreType.DMA for _ in range(NBUF)],
            [pltpu.SemaphoreType.DMA for _ in range(NBUF)],
            [pltpu.SemaphoreType.DMA for _ in range(NBUF)],
        ],
    )(rt, ct, row_g, col_g)

    msg = pl.pallas_call(
        _edge_body,
        grid=(KE,),
        in_specs=[
            pl.BlockSpec((BE, DW), lambda i: (i, 0)),
            pl.BlockSpec((BE, 8), lambda i: (i, 0)),
            pl.BlockSpec((1, M), lambda i: (0, 0)),
            pl.BlockSpec((M, 8), lambda i: (0, 0)),
            pl.BlockSpec((1, M), lambda i: (0, 0)),
            pl.BlockSpec((M, M), lambda i: (0, 0)),
            pl.BlockSpec((1, M), lambda i: (0, 0)),
            pl.BlockSpec((M, M), lambda i: (0, 0)),
            pl.BlockSpec((1, M), lambda i: (0, 0)),
        ],
        out_specs=pl.BlockSpec((BE, DW), lambda i: (i, 0)),
        out_shape=jax.ShapeDtypeStruct((E_pad, DW), f32),
    )(u, ea8, wrad, wea, be1, W_e2, be2, watt_rep, batt_rep)

    msum2 = pl.kernel(
        functools.partial(_scatter_body, K),
        out_type=jax.ShapeDtypeStruct((NC, NSEG, DW), f32),
        mesh=mesh,
        scratch_types=[
            pltpu.VMEM((K, CH), jnp.int32),
            pltpu.VMEM((CH, DW), f32),
            pltpu.VMEM_SHARED((NSEG, DW), f32),
        ],
    )(msg, sidx)

    out = pl.pallas_call(
        _node_body,
        grid=(N // BN,),
        in_specs=[
            pl.BlockSpec((BN, D), lambda i: (i, 0)),
            pl.BlockSpec((NC, BN, DW), lambda i: (0, i, 0)),
            pl.BlockSpec((M, D), lambda i: (0, 0)),
            pl.BlockSpec((M, M), lambda i: (0, 0)),
            pl.BlockSpec((1, M), lambda i: (0, 0)),
            pl.BlockSpec((D, M), lambda i: (0, 0)),
            pl.BlockSpec((1, D), lambda i: (0, 0)),
        ],
        out_specs=pl.BlockSpec((BN, D), lambda i: (i, 0)),
        out_shape=jax.ShapeDtypeStruct((N, D), f32),
    )(h, msum2, wh1a, wh1b, bh1, W_h2, bh2)
    return out


# speedup vs baseline: 1.7980x; 1.0524x over previous
import functools

import jax
import jax.numpy as jnp
from jax import lax
from jax.experimental import pallas as pl
from jax.experimental.pallas import tpu as pltpu
from jax.experimental.pallas import tpu_sc as plsc

N = 10000
D = 128
M = 64
DW = 128
PW = 64
CH = 128
NC = 2
NS = 16
NW = NC * NS
BN = 1000
BE2 = 2048
NSEG = 10112
RPT = NSEG // NS
MASKHI = -65536


def _silu(v):
    return v * jax.nn.sigmoid(v)


def _dg(a, b):
    return lax.dot_general(a, b, (((1,), (1,)), ((), ())),
                           preferred_element_type=jnp.float32)


def _table_body(h_ref, xp_ref, wr_ref, wc_ref, rt_ref, ct_ref):
    hh = h_ref[...]
    xp = xp_ref[...]
    for out_ref, w_ref, sgn in ((rt_ref, wr_ref, 1.0), (ct_ref, wc_ref, -1.0)):
        p = _dg(hh, w_ref[...])
        pbits = lax.shift_right_logical(
            lax.bitcast_convert_type(p, jnp.int32), 16)
        xbits = lax.bitcast_convert_type(sgn * xp, jnp.int32) & MASKHI
        out_ref[...] = pbits | xbits


def _edge_body(ur_ref, uc_ref, ea_ref, bdones_ref, wrad_ref, wea_ref, be1_ref,
               we2_ref, be2_ref, watt_ref, batt_ref, out_ref):
    ur = ur_ref[...]
    uc = uc_ref[...]

    def _lo(v):
        return lax.bitcast_convert_type(lax.shift_left(v, 16), jnp.float32)

    def _hi(v):
        return lax.bitcast_convert_type(v & MASKHI, jnp.float32)

    pq = _lo(ur) + _lo(uc)
    xd = _hi(ur) + _hi(uc)
    radb = jnp.sqrt(_dg(xd * xd, bdones_ref[...]))
    z = _silu(pq + radb * wrad_ref[...] + _dg(ea_ref[...], wea_ref[...])
              + be1_ref[...])
    m = _silu(_dg(z, we2_ref[...]) + be2_ref[...])
    att = jax.nn.sigmoid(_dg(m, watt_ref[...]) + batt_ref[...])
    out_ref[...] = m * att


def _node_body(h_ref, ms_ref, wh1a_ref, wh1b_ref, bh1_ref, wh2_ref, bh2_ref,
               out_ref):
    hh = h_ref[...]
    ms = ms_ref[0] + ms_ref[1]
    t = _silu(_dg(hh, wh1a_ref[...]) + _dg(ms, wh1b_ref[...]) + bh1_ref[...])
    out_ref[...] = hh + _dg(t, wh2_ref[...]) + bh2_ref[...]


def _gather_body(K, rt_hbm, ct_hbm, rowg_hbm, colg_hbm, ur_hbm, uc_hbm,
                 rowv, colv, rbufs, cbufs, ubrs, ubcs, semr, semc):
    c = lax.axis_index("c")
    s = lax.axis_index("s")
    wid = c * NS + s
    base = wid * (K * CH)
    pltpu.sync_copy(rowg_hbm.at[pl.ds(wid * K, K)], rowv)
    pltpu.sync_copy(colg_hbm.at[pl.ds(wid * K, K)], colv)

    def _repack(gb, ub):
        def body(i, _):
            for half in range(2):
                src = 2 * i + half
                for j in range(PW // 16):
                    ub[i, pl.ds(half * PW + j * 16, 16)] = (
                        gb[src, pl.ds(j * 16, 16)])
            return 0
        lax.fori_loop(0, CH // 2, body, 0, unroll=2)

    def _issue(k, i):
        pltpu.async_copy(rt_hbm.at[rowv.at[k]], rbufs[i], semr[i])
        pltpu.async_copy(ct_hbm.at[colv.at[k]], cbufs[i], semc[i])

    def _waitg(k, i):
        pltpu.make_async_copy(rt_hbm.at[rowv.at[k]], rbufs[i], semr[i]).wait()
        pltpu.make_async_copy(ct_hbm.at[colv.at[k]], cbufs[i], semc[i]).wait()

    def _emit(k, i):
        pltpu.sync_copy(ubrs[i],
                        ur_hbm.at[pl.ds((base + k * CH) // 2, CH // 2)])
        pltpu.sync_copy(ubcs[i],
                        uc_hbm.at[pl.ds((base + k * CH) // 2, CH // 2)])

    def _proc(k, i):
        _waitg(k, i)
        _repack(rbufs[i], ubrs[i])
        _repack(cbufs[i], ubcs[i])
        _emit(k, i)

    _issue(0, 0)

    def loop_body(k2, _):
        k0 = 2 * k2
        _issue(k0 + 1, 1)
        _proc(k0, 0)
        _issue(k0 + 2, 0)
        _proc(k0 + 1, 1)
        return 0

    lax.fori_loop(0, (K - 1) // 2, loop_body, 0)
    _proc(K - 1, 0)


_RPT_CHUNKS = [(o, min(128, RPT - o)) for o in range(0, RPT, 128)]


def _scatter_body(K, msg_hbm, sidx_hbm, out_hbm, sidxv, mbuf, sbuf, msum_sh):
    c = lax.axis_index("c")
    s = lax.axis_index("s")
    wid = c * NS + s
    base = wid * (K * CH)
    pltpu.sync_copy(sidx_hbm.at[pl.ds(wid * K, K)], sidxv)

    def zrow(i, _):
        for j in range(M // 16):
            sbuf[i, pl.ds(j * 16, 16)] = jnp.zeros((16,), jnp.float32)
        return 0
    lax.fori_loop(0, CH, zrow, 0)
    for off, sz in _RPT_CHUNKS:
        pltpu.sync_copy(sbuf.at[pl.ds(0, sz)],
                        msum_sh.at[pl.ds(s * RPT + off, sz)])
    plsc.subcore_barrier()

    def _unpack(i, _):
        for half in range(2):
            for j in range(M // 16):
                sbuf[2 * i + half, pl.ds(j * 16, 16)] = (
                    mbuf[i, pl.ds(half * M + j * 16, 16)])
        return 0

    def body(k, _):
        pltpu.sync_copy(msg_hbm.at[pl.ds((base + k * CH) // 2, CH // 2)],
                        mbuf)
        lax.fori_loop(0, CH // 2, _unpack, 0, unroll=2)
        pltpu.sync_copy(sbuf, msum_sh.at[sidxv.at[k]], add=True)
        return 0
    lax.fori_loop(0, K, body, 0)
    plsc.subcore_barrier()

    def _repack(i, _):
        for half in range(2):
            for j in range(M // 16):
                mbuf[i, pl.ds(half * M + j * 16, 16)] = (
                    sbuf[2 * i + half, pl.ds(j * 16, 16)])
        return 0

    for off, sz in _RPT_CHUNKS:
        pltpu.sync_copy(msum_sh.at[pl.ds(s * RPT + off, sz)],
                        sbuf.at[pl.ds(0, sz)])
        lax.fori_loop(0, sz // 2, _repack, 0, unroll=2)
        pltpu.sync_copy(mbuf.at[pl.ds(0, sz // 2)],
                        out_hbm.at[c, pl.ds((s * RPT + off) // 2, sz // 2)])


def kernel(x, h, edge_index, edge_attr, W_e1, b_e1, W_e2, b_e2, W_att, b_att,
           W_x1, b_x1, W_x2, b_x2, W_h1, b_h1, W_h2, b_h2):
    E = edge_index.shape[1]
    K = -(-E // (NW * CH))
    if K % 2 == 0:
        K += 1
    E_pad = NW * K * CH
    KE = E_pad // (2 * BE2)
    f32 = jnp.float32

    row = edge_index[0]
    col = edge_index[1]
    pad = E_pad - E
    row_g = jnp.concatenate([row, jnp.zeros((pad,), jnp.int32)])
    row_g = row_g.reshape(E_pad // CH, CH)
    col_g = jnp.concatenate([col, jnp.zeros((pad,), jnp.int32)])
    col_g = col_g.reshape(E_pad // CH, CH)
    sidx = jnp.concatenate([row, jnp.full((pad,), N, jnp.int32)])
    sidx = sidx.reshape(E_pad // CH, CH)
    ea2 = jnp.pad(edge_attr, ((0, pad), (0, 4))).reshape(E_pad // 2, 16)
    xp = jnp.pad(x, ((0, 0), (0, M - 3)))

    wr = W_e1[:, :D]
    wc = W_e1[:, D:2 * D]
    eye2 = jnp.eye(2, dtype=f32)
    bd_ones = jnp.kron(eye2, jnp.ones((M, M), f32))
    wrad2 = jnp.tile(W_e1[:, 2 * D:2 * D + 1].reshape(1, M), (1, 2))
    wea2 = jnp.kron(eye2, jnp.pad(W_e1[:, 2 * D + 1:], ((0, 0), (0, 4))))
    be1_2 = jnp.tile(b_e1.reshape(1, M), (1, 2))
    we2bd = jnp.kron(eye2, W_e2)
    be2_2 = jnp.tile(b_e2.reshape(1, M), (1, 2))
    watt2 = jnp.kron(eye2, jnp.tile(W_att, (M, 1)))
    batt2 = jnp.tile(b_att.reshape(1, 1), (1, 2 * M))
    wh1a = W_h1[:, :D]
    wh1b = W_h1[:, D:]
    bh1 = b_h1.reshape(1, M)
    bh2 = b_h2.reshape(1, D)

    rt, ct = pl.pallas_call(
        _table_body,
        grid=(N // BN,),
        in_specs=[
            pl.BlockSpec((BN, D), lambda i: (i, 0)),
            pl.BlockSpec((BN, M), lambda i: (i, 0)),
            pl.BlockSpec((M, D), lambda i: (0, 0)),
            pl.BlockSpec((M, D), lambda i: (0, 0)),
        ],
        out_specs=[
            pl.BlockSpec((BN, PW), lambda i: (i, 0)),
            pl.BlockSpec((BN, PW), lambda i: (i, 0)),
        ],
        out_shape=[
            jax.ShapeDtypeStruct((N, PW), jnp.int32),
            jax.ShapeDtypeStruct((N, PW), jnp.int32),
        ],
    )(h, xp, wr, wc)
    mesh = plsc.VectorSubcoreMesh(core_axis_name="c", subcore_axis_name="s")
    sc_params = pltpu.CompilerParams(use_tc_tiling_on_sc=False)
    u2r, u2c = pl.kernel(
        functools.partial(_gather_body, K),
        out_type=(jax.ShapeDtypeStruct((E_pad // 2, 2 * PW), jnp.int32),
                  jax.ShapeDtypeStruct((E_pad // 2, 2 * PW), jnp.int32)),
        mesh=mesh,
        compiler_params=sc_params,
        scratch_types=[
            pltpu.VMEM((K, CH), jnp.int32),
            pltpu.VMEM((K, CH), jnp.int32),
            [pltpu.VMEM((CH, PW), jnp.int32) for _ in range(2)],
            [pltpu.VMEM((CH, PW), jnp.int32) for _ in range(2)],
            [pltpu.VMEM((CH // 2, 2 * PW), jnp.int32) for _ in range(2)],
            [pltpu.VMEM((CH // 2, 2 * PW), jnp.int32) for _ in range(2)],
            [pltpu.SemaphoreType.DMA for _ in range(2)],
            [pltpu.SemaphoreType.DMA for _ in range(2)],
        ],
    )(rt, ct, row_g, col_g)

    msg2 = pl.pallas_call(
        _edge_body,
        grid=(KE,),
        in_specs=[
            pl.BlockSpec((BE2, 2 * PW), lambda i: (i, 0)),
            pl.BlockSpec((BE2, 2 * PW), lambda i: (i, 0)),
            pl.BlockSpec((BE2, 16), lambda i: (i, 0)),
            pl.BlockSpec((2 * M, 2 * M), lambda i: (0, 0)),
            pl.BlockSpec((1, 2 * M), lambda i: (0, 0)),
            pl.BlockSpec((2 * M, 16), lambda i: (0, 0)),
            pl.BlockSpec((1, 2 * M), lambda i: (0, 0)),
            pl.BlockSpec((2 * M, 2 * M), lambda i: (0, 0)),
            pl.BlockSpec((1, 2 * M), lambda i: (0, 0)),
            pl.BlockSpec((2 * M, 2 * M), lambda i: (0, 0)),
            pl.BlockSpec((1, 2 * M), lambda i: (0, 0)),
        ],
        out_specs=pl.BlockSpec((BE2, 2 * M), lambda i: (i, 0)),
        out_shape=jax.ShapeDtypeStruct((E_pad // 2, 2 * M), f32),
    )(u2r, u2c, ea2, bd_ones, wrad2, wea2, be1_2, we2bd, be2_2, watt2, batt2)

    msum2 = pl.kernel(
        functools.partial(_scatter_body, K),
        out_type=jax.ShapeDtypeStruct((NC, NSEG // 2, 2 * M), f32),
        mesh=mesh,
        compiler_params=sc_params,
        scratch_types=[
            pltpu.VMEM((K, CH), jnp.int32),
            pltpu.VMEM((CH // 2, 2 * M), f32),
            pltpu.VMEM((CH, M), f32),
            pltpu.VMEM_SHARED((NSEG, M), f32),
        ],
    )(msg2, sidx)
    msum = msum2.reshape(NC, NSEG, M)

    out = pl.pallas_call(
        _node_body,
        grid=(N // BN,),
        in_specs=[
            pl.BlockSpec((BN, D), lambda i: (i, 0)),
            pl.BlockSpec((NC, BN, M), lambda i: (0, i, 0)),
            pl.BlockSpec((M, D), lambda i: (0, 0)),
            pl.BlockSpec((M, M), lambda i: (0, 0)),
            pl.BlockSpec((1, M), lambda i: (0, 0)),
            pl.BlockSpec((D, M), lambda i: (0, 0)),
            pl.BlockSpec((1, D), lambda i: (0, 0)),
        ],
        out_specs=pl.BlockSpec((BN, D), lambda i: (i, 0)),
        out_shape=jax.ShapeDtypeStruct((N, D), f32),
    )(h, msum, wh1a, wh1b, bh1, W_h2, bh2)
    return out


# speedup vs baseline: 1.9454x; 1.0820x over previous
import functools

import jax
import jax.numpy as jnp
from jax import lax
from jax.experimental import pallas as pl
from jax.experimental.pallas import tpu as pltpu
from jax.experimental.pallas import tpu_sc as plsc

N = 10000
D = 128
M = 64
DW = 128
PW = 64
CH = 128
NC = 2
NS = 16
NW = NC * NS
BN = 1000
BE2 = 2048
NSEG = 10112
RPT = NSEG // NS
MASKHI = -65536


def _silu(v):
    return v * jax.nn.sigmoid(v)


def _dg(a, b):
    return lax.dot_general(a, b, (((1,), (1,)), ((), ())),
                           preferred_element_type=jnp.float32)


def _table_body(h_ref, xp_ref, wr_ref, wc_ref, rt_ref, ct_ref):
    hh = h_ref[...]
    xp = xp_ref[...]
    for out_ref, w_ref, sgn in ((rt_ref, wr_ref, 1.0), (ct_ref, wc_ref, -1.0)):
        p = _dg(hh, w_ref[...])
        pbits = lax.shift_right_logical(
            lax.bitcast_convert_type(p, jnp.int32), 16)
        xbits = lax.bitcast_convert_type(sgn * xp, jnp.int32) & MASKHI
        out_ref[...] = pbits | xbits


def _edge_body(ur_ref, uc_ref, ea_ref, bdones_ref, wrad_ref, wea_ref, be1_ref,
               we2_ref, be2_ref, watt_ref, batt_ref, out_ref):
    ur = ur_ref[...]
    uc = uc_ref[...]

    def _lo(v):
        return lax.bitcast_convert_type(lax.shift_left(v, 16), jnp.float32)

    def _hi(v):
        return lax.bitcast_convert_type(v & MASKHI, jnp.float32)

    pq = _lo(ur) + _lo(uc)
    xd = _hi(ur) + _hi(uc)
    radb = jnp.sqrt(_dg(xd * xd, bdones_ref[...]))
    z = _silu(pq + radb * wrad_ref[...] + _dg(ea_ref[...], wea_ref[...])
              + be1_ref[...])
    m = _silu(_dg(z, we2_ref[...]) + be2_ref[...])
    att = jax.nn.sigmoid(_dg(m, watt_ref[...]) + batt_ref[...])
    out_ref[...] = m * att


def _node_body(h_ref, ms_ref, wh1a_ref, wh1b_ref, bh1_ref, wh2_ref, bh2_ref,
               out_ref):
    hh = h_ref[...]
    ms = ms_ref[0] + ms_ref[1]
    t = _silu(_dg(hh, wh1a_ref[...]) + _dg(ms, wh1b_ref[...]) + bh1_ref[...])
    out_ref[...] = hh + _dg(t, wh2_ref[...]) + bh2_ref[...]


def _gather_body(K, rt_hbm, ct_hbm, rowg_hbm, colg_hbm, ur_hbm, uc_hbm,
                 rowv, colv, rbufs, cbufs, ubrs, ubcs, semr, semc):
    c = lax.axis_index("c")
    s = lax.axis_index("s")
    wid = c * NS + s
    base = wid * (K * CH)
    pltpu.sync_copy(rowg_hbm.at[pl.ds(wid * K, K)], rowv)
    pltpu.sync_copy(colg_hbm.at[pl.ds(wid * K, K)], colv)

    def _repack(gb, ub):
        def body(i, _):
            for half in range(2):
                src = 2 * i + half
                for j in range(PW // 16):
                    ub[i, pl.ds(half * PW + j * 16, 16)] = (
                        gb[src, pl.ds(j * 16, 16)])
            return 0
        lax.fori_loop(0, CH // 2, body, 0, unroll=2)

    def _issue(k, i):
        pltpu.async_copy(rt_hbm.at[rowv.at[k]], rbufs[i], semr[i])
        pltpu.async_copy(ct_hbm.at[colv.at[k]], cbufs[i], semc[i])

    def _waitg(k, i):
        pltpu.make_async_copy(rt_hbm.at[rowv.at[k]], rbufs[i], semr[i]).wait()
        pltpu.make_async_copy(ct_hbm.at[colv.at[k]], cbufs[i], semc[i]).wait()

    def _emit(k, i):
        pltpu.sync_copy(ubrs[i],
                        ur_hbm.at[pl.ds((base + k * CH) // 2, CH // 2)])
        pltpu.sync_copy(ubcs[i],
                        uc_hbm.at[pl.ds((base + k * CH) // 2, CH // 2)])

    def _proc(k, i):
        _waitg(k, i)
        _repack(rbufs[i], ubrs[i])
        _repack(cbufs[i], ubcs[i])
        _emit(k, i)

    _issue(0, 0)

    def loop_body(k2, _):
        k0 = 2 * k2
        _issue(k0 + 1, 1)
        _proc(k0, 0)
        _issue(k0 + 2, 0)
        _proc(k0 + 1, 1)
        return 0

    lax.fori_loop(0, (K - 1) // 2, loop_body, 0)
    _proc(K - 1, 0)


_RPT_CHUNKS = [(o, min(128, RPT - o)) for o in range(0, RPT, 128)]


def _scatter_body(K, msg_hbm, sidx_hbm, out_hbm, sidxv, mbufs, sbufs,
                  semm, msum_sh):
    c = lax.axis_index("c")
    s = lax.axis_index("s")
    wid = c * NS + s
    base = wid * (K * CH)
    pltpu.sync_copy(sidx_hbm.at[pl.ds(wid * K, K)], sidxv)

    def zrow(i, _):
        for j in range(M // 16):
            sbufs[0][i, pl.ds(j * 16, 16)] = jnp.zeros((16,), jnp.float32)
        return 0
    lax.fori_loop(0, CH, zrow, 0)
    for off, sz in _RPT_CHUNKS:
        pltpu.sync_copy(sbufs[0].at[pl.ds(0, sz)],
                        msum_sh.at[pl.ds(s * RPT + off, sz)])
    plsc.subcore_barrier()

    def _unpack(mb, sb):
        def body(i, _):
            for half in range(2):
                for j in range(M // 16):
                    sb[2 * i + half, pl.ds(j * 16, 16)] = (
                        mb[i, pl.ds(half * M + j * 16, 16)])
            return 0
        lax.fori_loop(0, CH // 2, body, 0, unroll=2)

    def _load(k, b):
        pltpu.async_copy(msg_hbm.at[pl.ds((base + k * CH) // 2, CH // 2)],
                         mbufs[b], semm[b])

    def _waitl(k, b):
        pltpu.make_async_copy(
            msg_hbm.at[pl.ds((base + k * CH) // 2, CH // 2)],
            mbufs[b], semm[b]).wait()

    def _proc(k, b):
        _waitl(k, b)
        _unpack(mbufs[b], sbufs[b])
        pltpu.sync_copy(sbufs[b], msum_sh.at[sidxv.at[k]], add=True)

    _load(0, 0)

    def body(k2, _):
        k0 = 2 * k2
        _load(k0 + 1, 1)
        _proc(k0, 0)
        _load(k0 + 2, 0)
        _proc(k0 + 1, 1)
        return 0
    lax.fori_loop(0, (K - 1) // 2, body, 0)
    _proc(K - 1, 0)
    plsc.subcore_barrier()

    def _repack(i, _):
        for half in range(2):
            for j in range(M // 16):
                mbufs[0][i, pl.ds(half * M + j * 16, 16)] = (
                    sbufs[0][2 * i + half, pl.ds(j * 16, 16)])
        return 0

    for off, sz in _RPT_CHUNKS:
        pltpu.sync_copy(msum_sh.at[pl.ds(s * RPT + off, sz)],
                        sbufs[0].at[pl.ds(0, sz)])
        lax.fori_loop(0, sz // 2, _repack, 0, unroll=2)
        pltpu.sync_copy(mbufs[0].at[pl.ds(0, sz // 2)],
                        out_hbm.at[c, pl.ds((s * RPT + off) // 2, sz // 2)])


def kernel(x, h, edge_index, edge_attr, W_e1, b_e1, W_e2, b_e2, W_att, b_att,
           W_x1, b_x1, W_x2, b_x2, W_h1, b_h1, W_h2, b_h2):
    E = edge_index.shape[1]
    K = -(-E // (NW * CH))
    if K % 2 == 0:
        K += 1
    E_pad = NW * K * CH
    KE = E_pad // (2 * BE2)
    f32 = jnp.float32

    row = edge_index[0]
    col = edge_index[1]
    pad = E_pad - E
    row_g = jnp.concatenate([row, jnp.zeros((pad,), jnp.int32)])
    row_g = row_g.reshape(E_pad // CH, CH)
    col_g = jnp.concatenate([col, jnp.zeros((pad,), jnp.int32)])
    col_g = col_g.reshape(E_pad // CH, CH)
    sidx = jnp.concatenate([row, jnp.full((pad,), N, jnp.int32)])
    sidx = sidx.reshape(E_pad // CH, CH)
    ea2 = jnp.pad(edge_attr, ((0, pad), (0, 4))).reshape(E_pad // 2, 16)
    xp = jnp.pad(x, ((0, 0), (0, M - 3)))

    wr = W_e1[:, :D]
    wc = W_e1[:, D:2 * D]
    eye2 = jnp.eye(2, dtype=f32)
    bd_ones = jnp.kron(eye2, jnp.ones((M, M), f32))
    wrad2 = jnp.tile(W_e1[:, 2 * D:2 * D + 1].reshape(1, M), (1, 2))
    wea2 = jnp.kron(eye2, jnp.pad(W_e1[:, 2 * D + 1:], ((0, 0), (0, 4))))
    be1_2 = jnp.tile(b_e1.reshape(1, M), (1, 2))
    we2bd = jnp.kron(eye2, W_e2)
    be2_2 = jnp.tile(b_e2.reshape(1, M), (1, 2))
    watt2 = jnp.kron(eye2, jnp.tile(W_att, (M, 1)))
    batt2 = jnp.tile(b_att.reshape(1, 1), (1, 2 * M))
    wh1a = W_h1[:, :D]
    wh1b = W_h1[:, D:]
    bh1 = b_h1.reshape(1, M)
    bh2 = b_h2.reshape(1, D)

    rt, ct = pl.pallas_call(
        _table_body,
        grid=(N // BN,),
        in_specs=[
            pl.BlockSpec((BN, D), lambda i: (i, 0)),
            pl.BlockSpec((BN, M), lambda i: (i, 0)),
            pl.BlockSpec((M, D), lambda i: (0, 0)),
            pl.BlockSpec((M, D), lambda i: (0, 0)),
        ],
        out_specs=[
            pl.BlockSpec((BN, PW), lambda i: (i, 0)),
            pl.BlockSpec((BN, PW), lambda i: (i, 0)),
        ],
        out_shape=[
            jax.ShapeDtypeStruct((N, PW), jnp.int32),
            jax.ShapeDtypeStruct((N, PW), jnp.int32),
        ],
    )(h, xp, wr, wc)
    mesh = plsc.VectorSubcoreMesh(core_axis_name="c", subcore_axis_name="s")
    sc_params = pltpu.CompilerParams(use_tc_tiling_on_sc=False)
    u2r, u2c = pl.kernel(
        functools.partial(_gather_body, K),
        out_type=(jax.ShapeDtypeStruct((E_pad // 2, 2 * PW), jnp.int32),
                  jax.ShapeDtypeStruct((E_pad // 2, 2 * PW), jnp.int32)),
        mesh=mesh,
        compiler_params=sc_params,
        scratch_types=[
            pltpu.VMEM((K, CH), jnp.int32),
            pltpu.VMEM((K, CH), jnp.int32),
            [pltpu.VMEM((CH, PW), jnp.int32) for _ in range(2)],
            [pltpu.VMEM((CH, PW), jnp.int32) for _ in range(2)],
            [pltpu.VMEM((CH // 2, 2 * PW), jnp.int32) for _ in range(2)],
            [pltpu.VMEM((CH // 2, 2 * PW), jnp.int32) for _ in range(2)],
            [pltpu.SemaphoreType.DMA for _ in range(2)],
            [pltpu.SemaphoreType.DMA for _ in range(2)],
        ],
    )(rt, ct, row_g, col_g)

    msg2 = pl.pallas_call(
        _edge_body,
        grid=(KE,),
        in_specs=[
            pl.BlockSpec((BE2, 2 * PW), lambda i: (i, 0)),
            pl.BlockSpec((BE2, 2 * PW), lambda i: (i, 0)),
            pl.BlockSpec((BE2, 16), lambda i: (i, 0)),
            pl.BlockSpec((2 * M, 2 * M), lambda i: (0, 0)),
            pl.BlockSpec((1, 2 * M), lambda i: (0, 0)),
            pl.BlockSpec((2 * M, 16), lambda i: (0, 0)),
            pl.BlockSpec((1, 2 * M), lambda i: (0, 0)),
            pl.BlockSpec((2 * M, 2 * M), lambda i: (0, 0)),
            pl.BlockSpec((1, 2 * M), lambda i: (0, 0)),
            pl.BlockSpec((2 * M, 2 * M), lambda i: (0, 0)),
            pl.BlockSpec((1, 2 * M), lambda i: (0, 0)),
        ],
        out_specs=pl.BlockSpec((BE2, 2 * M), lambda i: (i, 0)),
        out_shape=jax.ShapeDtypeStruct((E_pad // 2, 2 * M), f32),
    )(u2r, u2c, ea2, bd_ones, wrad2, wea2, be1_2, we2bd, be2_2, watt2, batt2)

    msum2 = pl.kernel(
        functools.partial(_scatter_body, K),
        out_type=jax.ShapeDtypeStruct((NC, NSEG // 2, 2 * M), f32),
        mesh=mesh,
        compiler_params=sc_params,
        scratch_types=[
            pltpu.VMEM((K, CH), jnp.int32),
            [pltpu.VMEM((CH // 2, 2 * M), f32) for _ in range(2)],
            [pltpu.VMEM((CH, M), f32) for _ in range(2)],
            [pltpu.SemaphoreType.DMA for _ in range(2)],
            pltpu.VMEM_SHARED((NSEG, M), f32),
        ],
    )(msg2, sidx)
    msum = msum2.reshape(NC, NSEG, M)

    out = pl.pallas_call(
        _node_body,
        grid=(N // BN,),
        in_specs=[
            pl.BlockSpec((BN, D), lambda i: (i, 0)),
            pl.BlockSpec((NC, BN, M), lambda i: (0, i, 0)),
            pl.BlockSpec((M, D), lambda i: (0, 0)),
            pl.BlockSpec((M, M), lambda i: (0, 0)),
            pl.BlockSpec((1, M), lambda i: (0, 0)),
            pl.BlockSpec((D, M), lambda i: (0, 0)),
            pl.BlockSpec((1, D), lambda i: (0, 0)),
        ],
        out_specs=pl.BlockSpec((BN, D), lambda i: (i, 0)),
        out_shape=jax.ShapeDtypeStruct((N, D), f32),
    )(h, msum, wh1a, wh1b, bh1, W_h2, bh2)
    return out
